# baseline (speedup 1.0000x reference)
import jax
import jax.numpy as jnp
from jax import lax
from jax.experimental import pallas as pl
from jax.experimental.pallas import tpu as pltpu
from jax.experimental.pallas import tpu_sc as plsc

_B, _N, _M = 128, 20000, 100
_NW = 32
_BPW = _B // _NW
_HB = 4096
_CAP = 256
_IMIN_PY = -(2 ** 31)
_UF = 5


def _scalar(v):
    return lax.squeeze(lax.slice(v, (0,), (1,)), (0,))


def _sc_body(preds_hbm, tgts_hbm, out_hbm, pred_v, key_v, hist_v,
             ckey_v, cidx_v, tgt_v, loss_v):
    _IMIN = jnp.int32(_IMIN_PY)
    lanes = lax.iota(jnp.int32, 16)
    ones_i = jnp.ones((16,), jnp.int32)
    zeros_i = jnp.zeros((16,), jnp.int32)
    imin_v = jnp.full((16,), _IMIN)
    wid = lax.axis_index("s") * 2 + lax.axis_index("c")

    def per_batch(j, acc_vec):
        bb = wid * _BPW + j
        pltpu.sync_copy(preds_hbm.at[pl.ds(bb * _N * 5, _N * 5)], pred_v)
        pltpu.sync_copy(tgts_hbm.at[pl.ds(bb * _M * 4, _M * 4)], tgt_v)

        def clr_h(i, _):
            for u in range(4):
                hist_v[pl.ds((i * 4 + u) * 16, 16)] = zeros_i
            return 0
        lax.fori_loop(0, _HB // 64, clr_h, 0)

        for c in range(_CAP // 16):
            ckey_v[pl.ds(c * 16, 16)] = imin_v

        def hist_pass(i, _):
            for u in range(_UF):
                i2 = i * _UF + u
                rows = i2 * 16 + lanes
                conf = plsc.load_gather(pred_v, [rows * 5 + 4])
                bits = plsc.bitcast(conf, jnp.int32)
                key = bits ^ ((bits >> 31) & jnp.int32(0x7FFFFFFF))
                key_v[pl.ds(i2 * 16, 16)] = key
                bucket = (key >> 20) + 2048
                plsc.addupdate_scatter(hist_v, [bucket], ones_i)
            return 0
        lax.fori_loop(0, _N // (16 * _UF), hist_pass, 0)

        def scan_pass(i, carry):
            before, bch, beforeh, found = carry
            for u in range(4):
                base = _HB - (i * 4 + u + 1) * 16
                chunk = hist_v[pl.ds(base, 16)]
                s = jnp.sum(chunk)
                crossing = jnp.logical_and(jnp.logical_not(found),
                                           (before + s) >= _M)
                bch = jnp.where(crossing, base, bch)
                beforeh = jnp.where(crossing, before, beforeh)
                found = jnp.logical_or(found, crossing)
                before = before + s
            return before, bch, beforeh, found
        _, bch, beforeh, _ = lax.fori_loop(
            0, _HB // 64, scan_pass,
            (jnp.int32(0), jnp.int32(0), jnp.int32(0), jnp.bool_(False)))
        wchunk = hist_v[pl.ds(bch, 16)]
        csum = plsc.cumsum(lax.rev(wchunk, (0,)))
        jlane = _scalar(plsc.all_reduce_ffs((beforeh + csum) >= _M))
        thr = bch + 15 - jlane

        def compress_pass(i, off):
            for u in range(_UF):
                i2 = i * _UF + u
                key = key_v[pl.ds(i2 * 16, 16)]
                mask = ((key >> 20) + 2048) >= thr
                offc = jnp.minimum(off, _CAP - 16)
                plsc.store_compressed(ckey_v.at[pl.ds(offc, 16)], key, mask=mask)
                plsc.store_compressed(cidx_v.at[pl.ds(offc, 16)],
                                      i2 * 16 + lanes, mask=mask)
                off = off + _scalar(plsc.all_reduce_population_count(mask))
            return off
        lax.fori_loop(0, _N // (16 * _UF), compress_pass, jnp.int32(0))

        cmv = imin_v
        for c in range(_CAP // 16):
            cm = jnp.max(ckey_v[pl.ds(c * 16, 16)])
            cmv = jnp.where(lanes == c, cm, cmv)

        mask4 = lanes < 4
        colsc = jnp.minimum(lanes, 4)
        tcol = jnp.minimum(lanes, 3)

        def extract(r, carry):
            acc, cmv = carry
            m = jnp.max(cmv)
            bc = _scalar(plsc.all_reduce_ffs(cmv == m))
            wchunk = ckey_v[pl.ds(bc * 16, 16)]
            lane = _scalar(plsc.all_reduce_ffs(wchunk == m))
            p = bc * 16 + lane
            pvec = jnp.full((16,), p)
            cidx = plsc.load_gather(cidx_v, [pvec])
            box = plsc.load_gather(pred_v, [cidx * 5 + colsc])
            tgt = plsc.load_gather(tgt_v, [r * 4 + tcol])
            d = box - tgt
            acc = acc + jnp.sum(jnp.where(mask4, d * d, jnp.float32(0.0)))
            nchunk = jnp.where(lanes == lane, imin_v, wchunk)
            ckey_v[pl.ds(bc * 16, 16)] = nchunk
            cmv = jnp.where(lanes == bc, jnp.max(nchunk), cmv)
            return acc, cmv
        acc, _ = lax.fori_loop(0, _M, extract, (jnp.float32(0.0), cmv))

        return jnp.where(lanes == j, acc, acc_vec)

    acc_vec = lax.fori_loop(0, _BPW, per_batch, jnp.zeros((16,), jnp.float32))
    loss_v[...] = acc_vec
    pltpu.sync_copy(loss_v, out_hbm.at[pl.ds(wid * 16, 16)])


def kernel(preds, targets):
    b = preds.shape[0]
    mesh = plsc.VectorSubcoreMesh(core_axis_name="c", subcore_axis_name="s")
    sc_call = pl.kernel(
        _sc_body,
        mesh=mesh,
        compiler_params=pltpu.CompilerParams(needs_layout_passes=False),
        out_type=jax.ShapeDtypeStruct((_NW * 16,), jnp.float32),
        scratch_types=[
            pltpu.VMEM((_N * 5,), jnp.float32),
            pltpu.VMEM((_N,), jnp.int32),
            pltpu.VMEM((_HB,), jnp.int32),
            pltpu.VMEM((_CAP,), jnp.int32),
            pltpu.VMEM((_CAP,), jnp.int32),
            pltpu.VMEM((_M * 4,), jnp.float32),
            pltpu.VMEM((16,), jnp.float32),
        ],
    )
    out = sc_call(preds.reshape(-1), targets.reshape(-1))
    return jnp.sum(out) / (b * _M * 4)


# speedup vs baseline: 4.4318x; 4.4318x over previous
import jax
import jax.numpy as jnp
from jax import lax
from jax.experimental import pallas as pl
from jax.experimental.pallas import tpu as pltpu
from jax.experimental.pallas import tpu_sc as plsc

_B, _N, _M = 128, 20000, 100
_NW = 32
_BPW = _B // _NW
_HB = 4096
_CAP = 256
_IMIN_PY = -(2 ** 31)
_UF = 5


def _scalar(v):
    return lax.squeeze(lax.slice(v, (0,), (1,)), (0,))


def _sc_body(preds_hbm, tgts_hbm, out_hbm, pred_v, key_v, hist_v,
             ckey_v, cidx_v, tgt_v, loss_v):
    _IMIN = jnp.int32(_IMIN_PY)
    lanes = lax.iota(jnp.int32, 16)
    ones_i = jnp.ones((16,), jnp.int32)
    zeros_i = jnp.zeros((16,), jnp.int32)
    imin_v = jnp.full((16,), _IMIN)
    wid = lax.axis_index("s") * 2 + lax.axis_index("c")

    def per_batch(j, acc_vec):
        bb = wid * _BPW + j
        pltpu.sync_copy(preds_hbm.at[bb], pred_v)
        pltpu.sync_copy(tgts_hbm.at[bb], tgt_v)

        def clr_h(i, _):
            for u in range(4):
                hist_v[pl.ds((i * 4 + u) * 16, 16)] = zeros_i
            return 0
        lax.fori_loop(0, _HB // 64, clr_h, 0)

        for c in range(_CAP // 16):
            ckey_v[pl.ds(c * 16, 16)] = imin_v

        def hist_pass(i, _):
            for u in range(_UF):
                i2 = i * _UF + u
                rows = i2 * 16 + lanes
                conf = plsc.load_gather(pred_v, [rows * 5 + 4])
                bits = plsc.bitcast(conf, jnp.int32)
                key = bits ^ ((bits >> 31) & jnp.int32(0x7FFFFFFF))
                key_v[pl.ds(i2 * 16, 16)] = key
                bucket = (key >> 20) + 2048
                plsc.addupdate_scatter(hist_v, [bucket], ones_i)
            return 0
        lax.fori_loop(0, _N // (16 * _UF), hist_pass, 0)

        def scan_pass(i, carry):
            before, bch, beforeh, found = carry
            for u in range(4):
                base = _HB - (i * 4 + u + 1) * 16
                chunk = hist_v[pl.ds(base, 16)]
                s = jnp.sum(chunk)
                crossing = jnp.logical_and(jnp.logical_not(found),
                                           (before + s) >= _M)
                bch = jnp.where(crossing, base, bch)
                beforeh = jnp.where(crossing, before, beforeh)
                found = jnp.logical_or(found, crossing)
                before = before + s
            return before, bch, beforeh, found
        _, bch, beforeh, _ = lax.fori_loop(
            0, _HB // 64, scan_pass,
            (jnp.int32(0), jnp.int32(0), jnp.int32(0), jnp.bool_(False)))
        wchunk = hist_v[pl.ds(bch, 16)]
        csum = plsc.cumsum(lax.rev(wchunk, (0,)))
        jlane = _scalar(plsc.all_reduce_ffs((beforeh + csum) >= _M))
        thr = bch + 15 - jlane

        def compress_pass(i, off):
            for u in range(_UF):
                i2 = i * _UF + u
                key = key_v[pl.ds(i2 * 16, 16)]
                mask = ((key >> 20) + 2048) >= thr
                offc = jnp.minimum(off, _CAP - 16)
                plsc.store_compressed(ckey_v.at[pl.ds(offc, 16)], key, mask=mask)
                plsc.store_compressed(cidx_v.at[pl.ds(offc, 16)],
                                      i2 * 16 + lanes, mask=mask)
                off = off + _scalar(plsc.all_reduce_population_count(mask))
            return off
        lax.fori_loop(0, _N // (16 * _UF), compress_pass, jnp.int32(0))

        cmv = imin_v
        for c in range(_CAP // 16):
            cm = jnp.max(ckey_v[pl.ds(c * 16, 16)])
            cmv = jnp.where(lanes == c, cm, cmv)

        mask4 = lanes < 4
        colsc = jnp.minimum(lanes, 4)
        tcol = jnp.minimum(lanes, 3)

        def extract(r, carry):
            acc, cmv = carry
            m = jnp.max(cmv)
            bc = _scalar(plsc.all_reduce_ffs(cmv == m))
            wchunk = ckey_v[pl.ds(bc * 16, 16)]
            lane = _scalar(plsc.all_reduce_ffs(wchunk == m))
            p = bc * 16 + lane
            pvec = jnp.full((16,), p)
            cidx = plsc.load_gather(cidx_v, [pvec])
            box = plsc.load_gather(pred_v, [cidx * 5 + colsc])
            tgt = plsc.load_gather(tgt_v, [r * 4 + tcol])
            d = box - tgt
            acc = acc + jnp.sum(jnp.where(mask4, d * d, jnp.float32(0.0)))
            nchunk = jnp.where(lanes == lane, imin_v, wchunk)
            ckey_v[pl.ds(bc * 16, 16)] = nchunk
            cmv = jnp.where(lanes == bc, jnp.max(nchunk), cmv)
            return acc, cmv
        acc, _ = lax.fori_loop(0, _M, extract, (jnp.float32(0.0), cmv))

        return jnp.where(lanes == j, acc, acc_vec)

    acc_vec = lax.fori_loop(0, _BPW, per_batch, jnp.zeros((16,), jnp.float32))
    loss_v[...] = acc_vec
    pltpu.sync_copy(loss_v, out_hbm.at[wid])


def kernel(preds, targets):
    b = preds.shape[0]
    mesh = plsc.VectorSubcoreMesh(core_axis_name="c", subcore_axis_name="s")
    sc_call = pl.kernel(
        _sc_body,
        mesh=mesh,
        compiler_params=pltpu.CompilerParams(needs_layout_passes=False, use_tc_tiling_on_sc=True),
        out_type=jax.ShapeDtypeStruct((_NW, 16), jnp.float32),
        scratch_types=[
            pltpu.VMEM((_N * 5,), jnp.float32),
            pltpu.VMEM((_N,), jnp.int32),
            pltpu.VMEM((_HB,), jnp.int32),
            pltpu.VMEM((_CAP,), jnp.int32),
            pltpu.VMEM((_CAP,), jnp.int32),
            pltpu.VMEM((_M * 4,), jnp.float32),
            pltpu.VMEM((16,), jnp.float32),
        ],
    )
    out = sc_call(preds.reshape(b, _N * 5), targets.reshape(b, _M * 4))
    return jnp.sum(out) / (b * _M * 4)
